# Initial kernel scaffold; baseline (speedup 1.0000x reference)
import jax
import jax.numpy as jnp
from jax.experimental import pallas as pl


def kernel(feature_a, feature_b, edges, W, b, a):
    raise NotImplementedError("write your pallas kernel here")


# speedup vs baseline: 4.7913x; 4.7913x over previous
import dataclasses
import functools

import jax
import jax.numpy as jnp
from jax import lax
from jax.experimental import pallas as pl
from jax.experimental.pallas import tpu as pltpu
from jax.experimental.pallas import tpu_sc as plsc

N_NODES = 10000
N_EDGES = 320000
D = 128

N_PAD = 10240
E_PAD = 327680
CHUNK = 128
CPW = E_PAD // 32 // CHUNK
SUPER = 8
ROWS_PER_TILE = N_PAD // 16


def _prep_body(fa_ref, fb_ref, w_ref, b_ref, a1_ref, a2_ref,
               ne_ref, sa_ref, sb_ref):
    ne = jnp.dot(fb_ref[...], w_ref[...],
                 preferred_element_type=jnp.float32,
                 precision=lax.Precision.HIGHEST) + b_ref[...]
    ne_ref[...] = ne
    sa_ref[...] = jnp.dot(fa_ref[...], a1_ref[...],
                          preferred_element_type=jnp.float32,
                          precision=lax.Precision.HIGHEST)
    sb_ref[...] = jnp.dot(ne, a2_ref[...],
                          preferred_element_type=jnp.float32,
                          precision=lax.Precision.HIGHEST)


_prep = pl.pallas_call(
    _prep_body,
    out_shape=(
        jax.ShapeDtypeStruct((N_PAD, D), jnp.float32),
        jax.ShapeDtypeStruct((N_PAD, 1), jnp.float32),
        jax.ShapeDtypeStruct((N_PAD, 1), jnp.float32),
    ),
)


def _edge_body(ne_hbm, sa_hbm, sb_hbm, src_hbm, dst_hbm,
               acc_out, rs_out,
               srcb, dstb, rows, val, sa_v, sb_v, acc_sh, rs_sh):
    c = lax.axis_index("c")
    s = lax.axis_index("s")

    pltpu.sync_copy(sa_hbm, sa_v)
    pltpu.sync_copy(sb_hbm, sb_v)
    base = (c * 16 + s) * CPW

    @pl.loop(0, CHUNK)
    def _(r):
        for k in range(D // 16):
            rows[r, pl.ds(k * 16, 16)] = jnp.zeros((16,), jnp.float32)

    for k in range(CHUNK // 16):
        val[pl.ds(k * 16, 16)] = jnp.zeros((16,), jnp.float32)

    for i in range(ROWS_PER_TILE // CHUNK):
        r0 = s * ROWS_PER_TILE + i * CHUNK
        pltpu.sync_copy(rows, acc_sh.at[pl.ds(r0, CHUNK)])
        pltpu.sync_copy(val, rs_sh.at[pl.ds(r0, CHUNK)])
    plsc.subcore_barrier()

    @pl.loop(0, CPW // SUPER)
    def _(jo):
        pltpu.sync_copy(src_hbm.at[pl.ds(base + jo * SUPER, SUPER)], srcb)
        pltpu.sync_copy(dst_hbm.at[pl.ds(base + jo * SUPER, SUPER)], dstb)

        @pl.loop(0, SUPER)
        def _(j):
            _edge_chunk(ne_hbm, srcb, dstb, rows, val, sa_v, sb_v,
                        acc_sh, rs_sh, j)

    plsc.subcore_barrier()
    for i in range(ROWS_PER_TILE // CHUNK):
        r0 = s * ROWS_PER_TILE + i * CHUNK
        pltpu.sync_copy(acc_sh.at[pl.ds(r0, CHUNK)],
                        acc_out.at[c].at[pl.ds(r0, CHUNK)])
        pltpu.sync_copy(rs_sh.at[pl.ds(r0, CHUNK)],
                        rs_out.at[c].at[pl.ds(r0, CHUNK)])


def _edge_chunk(ne_hbm, srcb, dstb, rows, val, sa_v, sb_v, acc_sh, rs_sh, j):
    pltpu.sync_copy(ne_hbm.at[dstb.at[j]], rows)
    for g in range(CHUNK // 16):
        sl = pl.ds(g * 16, 16)
        sv = srcb[j, sl]
        dv = dstb[j, sl]
        sc = plsc.load_gather(sa_v, [sv]) + plsc.load_gather(sb_v, [dv])
        elu = jnp.where(sc > 0, sc, 0.1 * (jnp.exp(sc) - 1.0))
        val[sl] = jnp.exp(elu)
    @pl.loop(0, CHUNK // 16)
    def _(g):
        val16 = val[pl.ds(g * 16, 16)]
        for jj in range(16):
            v = val16[jj]
            r = g * 16 + jj
            for k in range(D // 16):
                sl2 = pl.ds(k * 16, 16)
                rows[r, sl2] = rows[r, sl2] * v
    pltpu.sync_copy(val, rs_sh.at[srcb.at[j]], add=True)
    pltpu.sync_copy(rows, acc_sh.at[srcb.at[j]], add=True)


_sc_params = pltpu.CompilerParams()
if "needs_layout_passes" in pltpu.CompilerParams.__dataclass_fields__:
    _sc_params = dataclasses.replace(_sc_params, needs_layout_passes=False)

_edge = pl.kernel(
    _edge_body,
    compiler_params=_sc_params,
    out_type=(
        jax.ShapeDtypeStruct((2, N_PAD, D), jnp.float32),
        jax.ShapeDtypeStruct((2, N_PAD), jnp.float32),
    ),
    mesh=plsc.VectorSubcoreMesh(core_axis_name="c", subcore_axis_name="s"),
    scratch_types=[
        pltpu.VMEM((SUPER, CHUNK), jnp.int32),
        pltpu.VMEM((SUPER, CHUNK), jnp.int32),
        pltpu.VMEM((CHUNK, D), jnp.float32),
        pltpu.VMEM((CHUNK,), jnp.float32),
        pltpu.VMEM((N_PAD,), jnp.float32),
        pltpu.VMEM((N_PAD,), jnp.float32),
        pltpu.VMEM_SHARED((N_PAD, D), jnp.float32),
        pltpu.VMEM_SHARED((N_PAD,), jnp.float32),
    ],
)


def _fin_body(a0_ref, a1_ref, r0_ref, r1_ref, out_ref):
    rs = r0_ref[...] + r1_ref[...]
    rs = jnp.where(rs == 0.0, 1.0, rs)
    out_ref[...] = (a0_ref[...] + a1_ref[...]) / rs


_fin = pl.pallas_call(
    _fin_body,
    out_shape=jax.ShapeDtypeStruct((N_NODES, D), jnp.float32),
)


def kernel(feature_a, feature_b, edges, W, b, a):
    pad_n = N_PAD - N_NODES
    fa = jnp.pad(feature_a.astype(jnp.float32), ((0, pad_n), (0, 0)))
    fb = jnp.pad(feature_b.astype(jnp.float32), ((0, pad_n), (0, 0)))
    edges32 = edges.astype(jnp.int32)
    pad_e = E_PAD - N_EDGES
    src = jnp.concatenate(
        [edges32[:, 0], jnp.full((pad_e,), N_NODES, jnp.int32)])
    dst = jnp.concatenate([edges32[:, 1], jnp.zeros((pad_e,), jnp.int32)])
    src2 = src.reshape(E_PAD // CHUNK, CHUNK)
    dst2 = dst.reshape(E_PAD // CHUNK, CHUNK)

    ne, sa, sb = _prep(fa, fb, W.astype(jnp.float32),
                       b.astype(jnp.float32).reshape(1, D),
                       a[:D].astype(jnp.float32),
                       a[D:].astype(jnp.float32))
    acc, rs = _edge(ne, sa.reshape(-1), sb.reshape(-1), src2, dst2)
    return _fin(acc[0, :N_NODES], acc[1, :N_NODES],
                rs[0, :N_NODES].reshape(-1, 1),
                rs[1, :N_NODES].reshape(-1, 1))
